# baseline (speedup 1.0000x reference)
import functools

import jax
import jax.numpy as jnp
from jax.experimental import pallas as pl
from jax.experimental.pallas import tpu as pltpu

_N = 10000
_BM = 200
_GM = _N // _BM


def _gcn_body(adj_ref, x_ref, w0_ref, b0_ref, w1_ref, b1_ref, o_ref,
              xw0_s, hw1_s):
    i = pl.program_id(0)

    @pl.when(i == 0)
    def _prologue():
        xw0_s[...] = jnp.dot(
            x_ref[...].astype(jnp.bfloat16),
            w0_ref[...].astype(jnp.bfloat16),
            preferred_element_type=jnp.float32,
        ).astype(jnp.bfloat16)

    a = adj_ref[...].astype(jnp.bfloat16)

    @pl.when(i < _GM)
    def _layer0():
        acc = jnp.dot(a, xw0_s[...], preferred_element_type=jnp.float32)
        h = jnp.maximum(acc + b0_ref[...], 0.0)
        hw1_s[pl.ds(i * _BM, _BM), :] = jnp.dot(
            h.astype(jnp.bfloat16),
            w1_ref[...].astype(jnp.bfloat16),
            preferred_element_type=jnp.float32,
        ).astype(jnp.bfloat16)

    @pl.when(i >= _GM)
    def _layer1():
        o_ref[...] = (
            jnp.dot(a, hw1_s[...], preferred_element_type=jnp.float32)
            + b1_ref[...]
        )


@functools.partial(jax.jit, donate_argnums=())
def kernel(x, adj, W0, b0, W1, b1):
    n, d_in = x.shape
    d_hid = W0.shape[1]
    d_out = W1.shape[1]
    b0r = b0.reshape(1, d_hid)
    b1r = b1.reshape(1, d_out)

    out = pl.pallas_call(
        _gcn_body,
        grid=(2 * _GM,),
        in_specs=[
            pl.BlockSpec((_BM, n), lambda i: (i % _GM, 0)),
            pl.BlockSpec((n, d_in), lambda i: (0, 0)),
            pl.BlockSpec((d_in, d_hid), lambda i: (0, 0)),
            pl.BlockSpec((1, d_hid), lambda i: (0, 0)),
            pl.BlockSpec((d_hid, d_out), lambda i: (0, 0)),
            pl.BlockSpec((1, d_out), lambda i: (0, 0)),
        ],
        out_specs=pl.BlockSpec(
            (_BM, d_out), lambda i: (jnp.maximum(i - _GM, 0), 0)
        ),
        out_shape=jax.ShapeDtypeStruct((n, d_out), jnp.float32),
        scratch_shapes=[
            pltpu.VMEM((_N, 128), jnp.bfloat16),
            pltpu.VMEM((_N, 128), jnp.bfloat16),
        ],
        compiler_params=pltpu.CompilerParams(
            dimension_semantics=("arbitrary",),
        ),
    )(adj, x, W0, b0r, W1, b1r)

    return out


# speedup vs baseline: 1.0907x; 1.0907x over previous
import functools

import jax
import jax.numpy as jnp
from jax.experimental import pallas as pl
from jax.experimental.pallas import tpu as pltpu

_N = 10000
_BM = 400
_GM = _N // _BM


def _gcn_body(adj_ref, x_ref, w0_ref, b0_ref, w1_ref, b1_ref, o_ref,
              xw0_s, hw1_s):
    i = pl.program_id(0)

    @pl.when(i == 0)
    def _prologue():
        xw0_s[...] = jnp.dot(
            x_ref[...].astype(jnp.bfloat16),
            w0_ref[...].astype(jnp.bfloat16),
            preferred_element_type=jnp.float32,
        ).astype(jnp.bfloat16)

    a = adj_ref[...].astype(jnp.bfloat16)

    @pl.when(i < _GM)
    def _layer0():
        acc = jnp.dot(a, xw0_s[...], preferred_element_type=jnp.float32)
        h = jnp.maximum(acc + b0_ref[...], 0.0)
        hw1_s[pl.ds(i * _BM, _BM), :] = jnp.dot(
            h.astype(jnp.bfloat16),
            w1_ref[...].astype(jnp.bfloat16),
            preferred_element_type=jnp.float32,
        ).astype(jnp.bfloat16)

    @pl.when(i >= _GM)
    def _layer1():
        o_ref[...] = (
            jnp.dot(a, hw1_s[...], preferred_element_type=jnp.float32)
            + b1_ref[...]
        )


@functools.partial(jax.jit, donate_argnums=())
def kernel(x, adj, W0, b0, W1, b1):
    n, d_in = x.shape
    d_hid = W0.shape[1]
    d_out = W1.shape[1]
    b0r = b0.reshape(1, d_hid)
    b1r = b1.reshape(1, d_out)

    out = pl.pallas_call(
        _gcn_body,
        grid=(2 * _GM,),
        in_specs=[
            pl.BlockSpec((_BM, n), lambda i: (i % _GM, 0)),
            pl.BlockSpec((n, d_in), lambda i: (0, 0)),
            pl.BlockSpec((d_in, d_hid), lambda i: (0, 0)),
            pl.BlockSpec((1, d_hid), lambda i: (0, 0)),
            pl.BlockSpec((d_hid, d_out), lambda i: (0, 0)),
            pl.BlockSpec((1, d_out), lambda i: (0, 0)),
        ],
        out_specs=pl.BlockSpec(
            (_BM, d_out), lambda i: (jnp.maximum(i - _GM, 0), 0)
        ),
        out_shape=jax.ShapeDtypeStruct((n, d_out), jnp.float32),
        scratch_shapes=[
            pltpu.VMEM((_N, 128), jnp.bfloat16),
            pltpu.VMEM((_N, 128), jnp.bfloat16),
        ],
        compiler_params=pltpu.CompilerParams(
            dimension_semantics=("arbitrary",),
        ),
    )(adj, x, W0, b0r, W1, b1r)

    return out


# speedup vs baseline: 1.0944x; 1.0035x over previous
import functools

import jax
import jax.numpy as jnp
from jax.experimental import pallas as pl
from jax.experimental.pallas import tpu as pltpu

_N = 10000
_H = _N // 2
_BM = 200
_GM = _H // _BM


def _gcn_body(adj_t_ref, adj_b_ref, x_ref, w0_ref, b0_ref, w1_ref, b1_ref,
              o_ref, xw0_s, hw1_s, hw1bf_s):
    i = pl.program_id(0)

    @pl.when(i == 0)
    def _prologue():
        xw0_s[...] = jnp.dot(
            x_ref[...].astype(jnp.bfloat16),
            w0_ref[...].astype(jnp.bfloat16),
            preferred_element_type=jnp.float32,
        ).astype(jnp.bfloat16)

    a_t = adj_t_ref[0].astype(jnp.bfloat16)
    a_b = adj_b_ref[0].astype(jnp.bfloat16)

    @pl.when(i < _GM)
    def _layer0():
        for a, base in ((a_t, 0), (a_b, _H)):
            acc = jnp.dot(a, xw0_s[...], preferred_element_type=jnp.float32)
            h = jnp.maximum(acc + b0_ref[...], 0.0)
            hw1_s[pl.ds(base + i * _BM, _BM), :] = jnp.dot(
                h.astype(jnp.bfloat16),
                w1_ref[...].astype(jnp.bfloat16),
                preferred_element_type=jnp.float32,
            )

    @pl.when(i == _GM)
    def _cast_hw1():
        hw1bf_s[...] = hw1_s[...].astype(jnp.bfloat16)

    @pl.when(i >= _GM)
    def _layer1():
        o_ref[0] = (
            jnp.dot(a_t, hw1bf_s[...], preferred_element_type=jnp.float32)
            + b1_ref[...]
        )
        o_ref[1] = (
            jnp.dot(a_b, hw1bf_s[...], preferred_element_type=jnp.float32)
            + b1_ref[...]
        )


@functools.partial(jax.jit, donate_argnums=())
def kernel(x, adj, W0, b0, W1, b1):
    n, d_in = x.shape
    d_hid = W0.shape[1]
    d_out = W1.shape[1]
    b0r = b0.reshape(1, d_hid)
    b1r = b1.reshape(1, d_out)
    adj3 = adj.reshape(2, _H, n)

    const = pl.Buffered(buffer_count=1)
    out3 = pl.pallas_call(
        _gcn_body,
        grid=(2 * _GM,),
        in_specs=[
            pl.BlockSpec((1, _BM, n), lambda i: (0, i % _GM, 0)),
            pl.BlockSpec((1, _BM, n), lambda i: (1, i % _GM, 0)),
            pl.BlockSpec((n, d_in), lambda i: (0, 0), pipeline_mode=const),
            pl.BlockSpec((d_in, d_hid), lambda i: (0, 0), pipeline_mode=const),
            pl.BlockSpec((1, d_hid), lambda i: (0, 0), pipeline_mode=const),
            pl.BlockSpec((d_hid, d_out), lambda i: (0, 0), pipeline_mode=const),
            pl.BlockSpec((1, d_out), lambda i: (0, 0), pipeline_mode=const),
        ],
        out_specs=pl.BlockSpec(
            (2, _BM, d_out), lambda i: (0, jnp.maximum(i - _GM, 0), 0)
        ),
        out_shape=jax.ShapeDtypeStruct((2, _H, d_out), jnp.float32),
        scratch_shapes=[
            pltpu.VMEM((_N, 128), jnp.bfloat16),
            pltpu.VMEM((_N, 128), jnp.float32),
            pltpu.VMEM((_N, 128), jnp.bfloat16),
        ],
        compiler_params=pltpu.CompilerParams(
            dimension_semantics=("arbitrary",),
        ),
    )(adj3, adj3, x, W0, b0r, W1, b1r)

    return out3.reshape(n, d_out)
